# baseline (speedup 1.0000x reference)
import functools

import jax
import jax.numpy as jnp
from jax import lax
from jax.experimental import pallas as pl
from jax.experimental.pallas import tpu as pltpu
from jax.experimental.pallas import tpu_sc as plsc

CH = 128
NBUF = 8
LOOKAHEAD = 4


def _pair_split(v, bv):
    return (v // (2 * bv)) * bv


@functools.lru_cache(maxsize=None)
def _make_repack(v, d, bv=1792):
    a = _pair_split(v, bv)
    grid = a // bv + 1
    tail = v - 2 * a
    th = tail // 2

    @functools.partial(
        pl.pallas_call,
        grid=(grid,),
        in_specs=[
            pl.BlockSpec((d, bv), lambda i: (0, i)),
            pl.BlockSpec((d, bv), lambda i: (0, i + (grid - 1))),
        ],
        out_specs=pl.BlockSpec((bv, 2 * d), lambda i: (i, 0)),
        out_shape=jax.ShapeDtypeStruct((a + th, 2 * d), jnp.float32),
    )
    def repack(lo_ref, hi_ref, o_ref):
        i = pl.program_id(0)

        @pl.when(i < grid - 1)
        def _():
            o_ref[:, :d] = lo_ref[...].T
            o_ref[:, d:] = hi_ref[...].T

        @pl.when(i == grid - 1)
        def _():
            o_ref[:th, :d] = hi_ref[:, :th].T
            o_ref[:th, d:] = hi_ref[:, th : 2 * th].T

    return repack


def _pair_index(x, v, bv=1792):
    a = _pair_split(v, bv)
    th = (v - 2 * a) // 2
    main = jnp.where(x < a, 2 * x, 2 * (x - a) + 1)
    u = x - 2 * a
    tail = 2 * a + jnp.where(u < th, 2 * u, 2 * (u - th) + 1)
    return jnp.where(x < 2 * a, main, tail)


@functools.lru_cache(maxsize=None)
def _make_lookup(nw, nchunk, d):
    mesh = plsc.VectorSubcoreMesh(core_axis_name="c", subcore_axis_name="s")
    nc = plsc.get_sparse_core_info().num_cores
    tot = nw * nchunk * CH

    @functools.partial(
        pl.kernel,
        mesh=mesh,
        out_type=jax.ShapeDtypeStruct((tot, 2 * d), jnp.float32),
        scratch_types=[
            pltpu.VMEM((nchunk, CH), jnp.int32),
            pltpu.VMEM((NBUF, CH, d), jnp.float32),
            pltpu.SemaphoreType.DMA((NBUF,)),
            pltpu.SemaphoreType.DMA((NBUF,)),
        ],
        compiler_params=pltpu.CompilerParams(use_tc_tiling_on_sc=False),
    )
    def lookup(idx_hbm, table_hbm, out_hbm, idx_v, rows_v, gsem, osem):
        wid = lax.axis_index("s") * nc + lax.axis_index("c")
        pltpu.sync_copy(idx_hbm.at[wid], idx_v)
        base = wid * (nchunk * CH)

        def gather(j, b):
            pltpu.async_copy(table_hbm.at[idx_v.at[j]], rows_v.at[b], gsem.at[b])

        def wait_gather(j, b):
            pltpu.make_async_copy(
                table_hbm.at[idx_v.at[j]], rows_v.at[b], gsem.at[b]
            ).wait()

        def copy_out(j, b):
            pltpu.async_copy(
                rows_v.at[b],
                out_hbm.at[pl.ds(base + j * CH, CH), pl.ds(0, d)],
                osem.at[b],
            )

        def wait_out(j, b):
            pltpu.make_async_copy(
                rows_v.at[b],
                out_hbm.at[pl.ds(base + j * CH, CH), pl.ds(0, d)],
                osem.at[b],
            ).wait()

        for j in range(LOOKAHEAD):
            gather(j, j % NBUF)

        def group(g, carry):
            for b in range(NBUF):
                j = g * NBUF + b
                bn = (b + LOOKAHEAD) % NBUF
                jn = j + LOOKAHEAD
                wait_gather(j, b)
                copy_out(j, b)

                @pl.when(jnp.logical_and(jn >= NBUF, jn < nchunk))
                def _():
                    wait_out(jn - NBUF, bn)

                @pl.when(jn < nchunk)
                def _():
                    gather(jn, bn)

            return carry

        lax.fori_loop(0, nchunk // NBUF, group, 0)

        for b in range(NBUF):
            j = nchunk - NBUF + b
            wait_out(j, j % NBUF)

    return lookup


def kernel(x, table):
    b, s = x.shape
    v, d = table.shape
    tot = b * s
    nw = 32
    nchunk = tot // (nw * CH)
    tt = table.T
    tp = _make_repack(v, d)(tt, tt).reshape(v, d)
    xr = _pair_index(x.reshape(nw, nchunk, CH).astype(jnp.int32), v)
    out = _make_lookup(nw, nchunk, d)(xr, tp)
    return out[:, :d].reshape(b, s, d)


# speedup vs baseline: 1.2129x; 1.2129x over previous
import functools

import jax
import jax.numpy as jnp
from jax import lax
from jax.experimental import pallas as pl
from jax.experimental.pallas import tpu as pltpu
from jax.experimental.pallas import tpu_sc as plsc

CH = 128
NBUF = 8
LOOKAHEAD = 4
BV = 7936


def _pair_split(v, bv):
    return (v // (2 * bv)) * bv


@functools.lru_cache(maxsize=None)
def _make_repack(v, d, bv=BV):
    a = _pair_split(v, bv)
    grid = a // bv + 1
    tail = v - 2 * a
    th = tail // 2

    @functools.partial(
        pl.pallas_call,
        grid=(grid,),
        in_specs=[
            pl.BlockSpec((d, bv), lambda i: (0, i)),
            pl.BlockSpec((d, bv), lambda i: (0, i + (grid - 1))),
        ],
        out_specs=pl.BlockSpec((bv, 2 * d), lambda i: (i, 0)),
        out_shape=jax.ShapeDtypeStruct((a + th, 2 * d), jnp.float32),
    )
    def repack(lo_ref, hi_ref, o_ref):
        i = pl.program_id(0)

        @pl.when(i < grid - 1)
        def _():
            o_ref[:, :d] = lo_ref[...].T
            o_ref[:, d:] = hi_ref[...].T

        @pl.when(i == grid - 1)
        def _():
            o_ref[:th, :d] = hi_ref[:, :th].T
            o_ref[:th, d:] = hi_ref[:, th : 2 * th].T

    return repack


def _pair_index(x, v, bv=BV):
    a = _pair_split(v, bv)
    th = (v - 2 * a) // 2
    main = jnp.where(x < a, 2 * x, 2 * (x - a) + 1)
    u = x - 2 * a
    tail = 2 * a + jnp.where(u < th, 2 * u, 2 * (u - th) + 1)
    return jnp.where(x < 2 * a, main, tail)


@functools.lru_cache(maxsize=None)
def _make_lookup(nw, nchunk, d):
    mesh = plsc.VectorSubcoreMesh(core_axis_name="c", subcore_axis_name="s")
    nc = plsc.get_sparse_core_info().num_cores
    tot = nw * nchunk * CH

    @functools.partial(
        pl.kernel,
        mesh=mesh,
        out_type=jax.ShapeDtypeStruct((tot, 2 * d), jnp.float32),
        scratch_types=[
            pltpu.VMEM((nchunk, CH), jnp.int32),
            pltpu.VMEM((NBUF, CH, d), jnp.float32),
            pltpu.SemaphoreType.DMA((NBUF,)),
            pltpu.SemaphoreType.DMA((NBUF,)),
        ],
        compiler_params=pltpu.CompilerParams(use_tc_tiling_on_sc=False),
    )
    def lookup(idx_hbm, table_hbm, out_hbm, idx_v, rows_v, gsem, osem):
        wid = lax.axis_index("s") * nc + lax.axis_index("c")
        pltpu.sync_copy(idx_hbm.at[wid], idx_v)
        base = wid * (nchunk * CH)

        def gather(j, b):
            pltpu.async_copy(table_hbm.at[idx_v.at[j]], rows_v.at[b], gsem.at[b])

        def wait_gather(j, b):
            pltpu.make_async_copy(
                table_hbm.at[idx_v.at[j]], rows_v.at[b], gsem.at[b]
            ).wait()

        def copy_out(j, b):
            pltpu.async_copy(
                rows_v.at[b],
                out_hbm.at[pl.ds(base + j * CH, CH), pl.ds(0, d)],
                osem.at[b],
            )

        def wait_out(j, b):
            pltpu.make_async_copy(
                rows_v.at[b],
                out_hbm.at[pl.ds(base + j * CH, CH), pl.ds(0, d)],
                osem.at[b],
            ).wait()

        for j in range(LOOKAHEAD):
            gather(j, j % NBUF)

        def group(g, carry):
            for b in range(NBUF):
                j = g * NBUF + b
                bn = (b + LOOKAHEAD) % NBUF
                jn = j + LOOKAHEAD
                wait_gather(j, b)
                copy_out(j, b)

                @pl.when(jnp.logical_and(jn >= NBUF, jn < nchunk))
                def _():
                    wait_out(jn - NBUF, bn)

                @pl.when(jn < nchunk)
                def _():
                    gather(jn, bn)

            return carry

        lax.fori_loop(0, nchunk // NBUF, group, 0)

        for b in range(NBUF):
            j = nchunk - NBUF + b
            wait_out(j, j % NBUF)

    return lookup


def kernel(x, table):
    b, s = x.shape
    v, d = table.shape
    tot = b * s
    nw = 32
    nchunk = tot // (nw * CH)
    tt = table.T
    tp = _make_repack(v, d)(tt, tt)
    tp = tp.reshape(2 * tp.shape[0], d)
    xr = _pair_index(x.reshape(nw, nchunk, CH).astype(jnp.int32), v)
    out = _make_lookup(nw, nchunk, d)(xr, tp)
    return out[:, :d].reshape(b, s, d)
